# baseline (speedup 1.0000x reference)
import functools

import jax
import jax.numpy as jnp
from jax import lax
from jax.experimental import pallas as pl
from jax.experimental.pallas import tpu as pltpu
from jax.experimental.pallas import tpu_sc as plsc

NC = 2
NS = 16
NW = NC * NS
LANES = 16
CH = 50


def _sc_degree(dst, n):
  e = dst.shape[0]
  epw = e // NW
  mesh = plsc.VectorSubcoreMesh(core_axis_name="c", subcore_axis_name="s")

  @functools.partial(
      pl.kernel,
      mesh=mesh,
      out_type=jax.ShapeDtypeStruct((NW, n), jnp.float32),
      compiler_params=pltpu.CompilerParams(needs_layout_passes=False, use_tc_tiling_on_sc=False),
      scratch_types=[
          pltpu.VMEM((n,), jnp.float32),
          pltpu.VMEM((epw,), jnp.int32),
          pltpu.SemaphoreType.DMA,
      ],
  )
  def deg_kernel(dst_hbm, out_hbm, degbuf, dstbuf, dsem):
    cid = lax.axis_index("c")
    sid = lax.axis_index("s")
    wid = sid * NC + cid

    pltpu.async_copy(dst_hbm.at[pl.ds(wid * epw, epw)], dstbuf, dsem)

    zero = jnp.zeros((LANES,), jnp.float32)

    def zbody(i, carry):
      degbuf[pl.ds(i * LANES, LANES)] = zero
      return carry

    lax.fori_loop(0, n // LANES, zbody, 0)

    pltpu.make_async_copy(dst_hbm.at[pl.ds(wid * epw, epw)], dstbuf,
                          dsem).wait()

    ones = jnp.ones((LANES,), jnp.float32)

    def body(j, carry):
      idx = dstbuf[pl.ds(j * LANES, LANES)]
      plsc.addupdate_scatter(degbuf, [idx], ones)
      return carry

    lax.fori_loop(0, epw // LANES, body, 0)
    pltpu.sync_copy(degbuf, out_hbm.at[wid])

  return deg_kernel(dst)


def _sc_scatter_rows(table, src2d, dst2d, n):
  d = table.shape[1]
  nch_total = src2d.shape[0]
  nch = nch_total // NW
  rows_per_tile = n // NS
  mesh = plsc.VectorSubcoreMesh(core_axis_name="c", subcore_axis_name="s")

  @functools.partial(
      pl.kernel,
      mesh=mesh,
      out_type=jax.ShapeDtypeStruct((NC, n, d), jnp.float32),
      compiler_params=pltpu.CompilerParams(needs_layout_passes=False, use_tc_tiling_on_sc=False),
      scratch_types=[
          pltpu.VMEM_SHARED((n, d), jnp.float32),
          pltpu.VMEM((nch, CH), jnp.int32),
          pltpu.VMEM((nch, CH), jnp.int32),
          pltpu.VMEM((4, CH, d), jnp.float32),
          [pltpu.SemaphoreType.DMA] * 4,
          [pltpu.SemaphoreType.DMA] * 4,
      ],
  )
  def scat_kernel(table_hbm, src_hbm, dst_hbm, out_hbm,
                  acc_sh, srcb, dstb, rowb, gsem, ssem):
    cid = lax.axis_index("c")
    sid = lax.axis_index("s")
    wid = sid * NC + cid

    zero = jnp.zeros((LANES,), jnp.float32)
    zsrc = rowb.at[0]

    def zbody(r, carry):
      for c in range(d // LANES):
        zsrc[r, pl.ds(c * LANES, LANES)] = zero
      return carry

    lax.fori_loop(0, CH, zbody, 0)

    row0 = sid * rows_per_tile
    ztail = rows_per_tile % CH
    for k in range(rows_per_tile // CH):
      pltpu.async_copy(zsrc, acc_sh.at[pl.ds(row0 + k * CH, CH)], ssem[0])
    pltpu.async_copy(zsrc.at[pl.ds(0, ztail)],
                     acc_sh.at[pl.ds(row0 + rows_per_tile - ztail, ztail)],
                     ssem[1])

    pltpu.sync_copy(src_hbm.at[pl.ds(wid * nch, nch)], srcb)
    pltpu.sync_copy(dst_hbm.at[pl.ds(wid * nch, nch)], dstb)

    for k in range(rows_per_tile // CH):
      pltpu.make_async_copy(zsrc, acc_sh.at[pl.ds(row0, CH)], ssem[0]).wait()
    pltpu.make_async_copy(
        zsrc.at[pl.ds(0, ztail)],
        acc_sh.at[pl.ds(row0 + rows_per_tile - ztail, ztail)],
        ssem[1]).wait()
    plsc.subcore_barrier()

    NBUF = 4
    bufs = [rowb.at[b] for b in range(NBUF)]
    ngrp = nch // NBUF

    for b in range(NBUF):
      pltpu.async_copy(table_hbm.at[srcb.at[b]], bufs[b], gsem[b])

    def body(k, carry):
      c = NBUF * k
      for b in range(NBUF):
        pltpu.make_async_copy(
            table_hbm.at[srcb.at[c + b]], bufs[b], gsem[b]).wait()
        pltpu.async_copy(bufs[b], acc_sh.at[dstb.at[c + b]], ssem[b],
                         add=True)
        pltpu.make_async_copy(bufs[b], acc_sh.at[dstb.at[c + b]],
                              ssem[b]).wait()
        pltpu.async_copy(table_hbm.at[srcb.at[c + NBUF + b]], bufs[b],
                         gsem[b])
      return carry

    lax.fori_loop(0, ngrp - 1, body, 0)
    cl = nch - NBUF
    for b in range(NBUF):
      pltpu.make_async_copy(
          table_hbm.at[srcb.at[cl + b]], bufs[b], gsem[b]).wait()
      pltpu.async_copy(bufs[b], acc_sh.at[dstb.at[cl + b]], ssem[b],
                       add=True)
    for b in range(NBUF):
      pltpu.make_async_copy(bufs[b], acc_sh.at[dstb.at[cl + b]],
                            ssem[b]).wait()
    plsc.subcore_barrier()

    pltpu.sync_copy(acc_sh.at[pl.ds(row0, rows_per_tile)],
                    out_hbm.at[cid, pl.ds(row0, rows_per_tile)])

  return scat_kernel(table, src2d, dst2d)


def _tc1(x, w1, disb, n, blk=1000):
  d = w1.shape[1]

  def body(x_ref, w_ref, disb_ref, g1_ref, h1s_ref):
    g = jnp.dot(x_ref[...], w_ref[...], preferred_element_type=jnp.float32)
    g1_ref[...] = g
    h1s_ref[...] = g * disb_ref[...]

  return pl.pallas_call(
      body,
      grid=(n // blk,),
      in_specs=[
          pl.BlockSpec((blk, x.shape[1]), lambda i: (i, 0)),
          pl.BlockSpec(w1.shape, lambda i: (0, 0)),
          pl.BlockSpec((blk, d), lambda i: (i, 0)),
      ],
      out_specs=[
          pl.BlockSpec((blk, d), lambda i: (i, 0)),
          pl.BlockSpec((blk, d), lambda i: (i, 0)),
      ],
      out_shape=[
          jax.ShapeDtypeStruct((n, d), jnp.float32),
          jax.ShapeDtypeStruct((n, d), jnp.float32),
      ],
  )(x, w1, disb)


def _tc2(s1a, s1b, g1, disb, b1, w2, n, blk=1000):
  d = w2.shape[1]

  def body(sa_ref, sb_ref, g1_ref, disb_ref, b1_ref, w2_ref, g2_ref, h2s_ref):
    dis = disb_ref[...]
    h = jnp.maximum(
        dis * (sa_ref[...] + sb_ref[...]) + dis * dis * g1_ref[...]
        + b1_ref[...], 0.0)
    g2 = jnp.dot(h, w2_ref[...], preferred_element_type=jnp.float32)
    g2_ref[...] = g2
    h2s_ref[...] = g2 * dis

  return pl.pallas_call(
      body,
      grid=(n // blk,),
      in_specs=[
          pl.BlockSpec((blk, d), lambda i: (i, 0)),
          pl.BlockSpec((blk, d), lambda i: (i, 0)),
          pl.BlockSpec((blk, d), lambda i: (i, 0)),
          pl.BlockSpec((blk, d), lambda i: (i, 0)),
          pl.BlockSpec((1, d), lambda i: (0, 0)),
          pl.BlockSpec(w2.shape, lambda i: (0, 0)),
      ],
      out_specs=[
          pl.BlockSpec((blk, d), lambda i: (i, 0)),
          pl.BlockSpec((blk, d), lambda i: (i, 0)),
      ],
      out_shape=[
          jax.ShapeDtypeStruct((n, d), jnp.float32),
          jax.ShapeDtypeStruct((n, d), jnp.float32),
      ],
  )(s1a, s1b, g1, disb, b1, w2)


def _tc3a(x, wp1b, bp1, n, blk=1000):
  din = x.shape[1]
  ph = wp1b.shape[1]

  def body(x_ref, wb_ref, bp1_ref, xp_ref):
    xp_ref[...] = (
        jnp.dot(x_ref[...], wb_ref[...], preferred_element_type=jnp.float32)
        + bp1_ref[...])

  return pl.pallas_call(
      body,
      grid=(n // blk,),
      in_specs=[
          pl.BlockSpec((blk, din), lambda i: (i, 0)),
          pl.BlockSpec(wp1b.shape, lambda i: (0, 0)),
          pl.BlockSpec((1, ph), lambda i: (0, 0)),
      ],
      out_specs=pl.BlockSpec((blk, ph), lambda i: (i, 0)),
      out_shape=jax.ShapeDtypeStruct((n, ph), jnp.float32),
  )(x, wp1b, bp1)


def _tc3(s2a, s2b, g2, disb, b2, xp, wp1a, wp2, bp2, n, blk=1000):
  d = g2.shape[1]
  pout = wp2.shape[1]

  def body(sa_ref, sb_ref, g2_ref, disb_ref, b2_ref, xp_ref,
           wa_ref, wp2_ref, bp2_ref, y_ref):
    dis = disb_ref[...]
    gnn = (dis * (sa_ref[...] + sb_ref[...]) + dis * dis * g2_ref[...]
           + b2_ref[...])
    p = jnp.maximum(
        jnp.dot(gnn, wa_ref[...], preferred_element_type=jnp.float32)
        + xp_ref[...], 0.0)
    y_ref[...] = (jnp.dot(p, wp2_ref[...], preferred_element_type=jnp.float32)
                  + bp2_ref[...])

  return pl.pallas_call(
      body,
      grid=(n // blk,),
      in_specs=[
          pl.BlockSpec((blk, d), lambda i: (i, 0)),
          pl.BlockSpec((blk, d), lambda i: (i, 0)),
          pl.BlockSpec((blk, d), lambda i: (i, 0)),
          pl.BlockSpec((blk, d), lambda i: (i, 0)),
          pl.BlockSpec((1, d), lambda i: (0, 0)),
          pl.BlockSpec((blk, wp1a.shape[1]), lambda i: (i, 0)),
          pl.BlockSpec(wp1a.shape, lambda i: (0, 0)),
          pl.BlockSpec(wp2.shape, lambda i: (0, 0)),
          pl.BlockSpec((1, pout), lambda i: (0, 0)),
      ],
      out_specs=pl.BlockSpec((blk, pout), lambda i: (i, 0)),
      out_shape=jax.ShapeDtypeStruct((n, pout), jnp.float32),
  )(s2a, s2b, g2, disb, b2, xp, wp1a, wp2, bp2)


def kernel(x, edge_index, W1, b1, W2, b2, Wp1, bp1, Wp2, bp2):
  n, din = x.shape
  e = edge_index.shape[1]
  d = W1.shape[1]

  src2d = edge_index[0].reshape(e // CH, CH)
  dst2d = edge_index[1].reshape(e // CH, CH)

  deg_parts = _sc_degree(edge_index[1], n)
  dis = lax.rsqrt(1.0 + jnp.sum(deg_parts, axis=0))
  disb = jnp.broadcast_to(dis[:, None], (n, d))

  g1, h1s = _tc1(x, W1, disb, n)
  s1 = _sc_scatter_rows(h1s, src2d, dst2d, n)
  g2, h2s = _tc2(s1[0], s1[1], g1, disb, b1.reshape(1, d), W2, n)
  s2 = _sc_scatter_rows(h2s, src2d, dst2d, n)
  xp = _tc3a(x, Wp1[d:], bp1.reshape(1, -1), n)
  y = _tc3(s2[0], s2[1], g2, disb, b2.reshape(1, d), xp,
           Wp1[:d], Wp2, bp2.reshape(1, -1), n)
  return y


# speedup vs baseline: 1.0033x; 1.0033x over previous
import functools

import jax
import jax.numpy as jnp
from jax import lax
from jax.experimental import pallas as pl
from jax.experimental.pallas import tpu as pltpu
from jax.experimental.pallas import tpu_sc as plsc

NC = 2
NS = 16
NW = NC * NS
LANES = 16
CH = 50


def _sc_degree(dst, n):
  e = dst.shape[0]
  epw = e // NW
  mesh = plsc.VectorSubcoreMesh(core_axis_name="c", subcore_axis_name="s")

  @functools.partial(
      pl.kernel,
      mesh=mesh,
      out_type=jax.ShapeDtypeStruct((NW, n), jnp.float32),
      compiler_params=pltpu.CompilerParams(needs_layout_passes=False, use_tc_tiling_on_sc=False),
      scratch_types=[
          pltpu.VMEM((n,), jnp.float32),
          pltpu.VMEM((epw,), jnp.int32),
      ],
  )
  def deg_kernel(dst_hbm, out_hbm, degbuf, dstbuf):
    cid = lax.axis_index("c")
    sid = lax.axis_index("s")
    wid = sid * NC + cid

    zero = jnp.zeros((LANES,), jnp.float32)

    def zbody(i, carry):
      degbuf[pl.ds(i * LANES, LANES)] = zero
      return carry

    lax.fori_loop(0, n // LANES, zbody, 0)

    pltpu.sync_copy(dst_hbm.at[pl.ds(wid * epw, epw)], dstbuf)

    ones = jnp.ones((LANES,), jnp.float32)

    def body(j, carry):
      idx = dstbuf[pl.ds(j * LANES, LANES)]
      plsc.addupdate_scatter(degbuf, [idx], ones)
      return carry

    lax.fori_loop(0, epw // LANES, body, 0)
    pltpu.sync_copy(degbuf, out_hbm.at[wid])

  return deg_kernel(dst)


def _sc_scatter_rows(table, src2d, dst2d, n):
  d = table.shape[1]
  nch_total = src2d.shape[0]
  nch = nch_total // NW
  rows_per_tile = n // NS
  mesh = plsc.VectorSubcoreMesh(core_axis_name="c", subcore_axis_name="s")

  @functools.partial(
      pl.kernel,
      mesh=mesh,
      out_type=jax.ShapeDtypeStruct((NC, n, d), jnp.float32),
      compiler_params=pltpu.CompilerParams(needs_layout_passes=False, use_tc_tiling_on_sc=False),
      scratch_types=[
          pltpu.VMEM_SHARED((n, d), jnp.float32),
          pltpu.VMEM((nch, CH), jnp.int32),
          pltpu.VMEM((nch, CH), jnp.int32),
          pltpu.VMEM((4, CH, d), jnp.float32),
          [pltpu.SemaphoreType.DMA] * 4,
          [pltpu.SemaphoreType.DMA] * 4,
      ],
  )
  def scat_kernel(table_hbm, src_hbm, dst_hbm, out_hbm,
                  acc_sh, srcb, dstb, rowb, gsem, ssem):
    cid = lax.axis_index("c")
    sid = lax.axis_index("s")
    wid = sid * NC + cid

    zero = jnp.zeros((LANES,), jnp.float32)
    zsrc = rowb.at[0]

    def zbody(r, carry):
      for c in range(d // LANES):
        zsrc[r, pl.ds(c * LANES, LANES)] = zero
      return carry

    lax.fori_loop(0, CH, zbody, 0)

    row0 = sid * rows_per_tile
    ztail = rows_per_tile % CH
    for k in range(rows_per_tile // CH):
      pltpu.async_copy(zsrc, acc_sh.at[pl.ds(row0 + k * CH, CH)], ssem[0])
    pltpu.async_copy(zsrc.at[pl.ds(0, ztail)],
                     acc_sh.at[pl.ds(row0 + rows_per_tile - ztail, ztail)],
                     ssem[1])

    pltpu.sync_copy(src_hbm.at[pl.ds(wid * nch, nch)], srcb)
    pltpu.sync_copy(dst_hbm.at[pl.ds(wid * nch, nch)], dstb)

    for k in range(rows_per_tile // CH):
      pltpu.make_async_copy(zsrc, acc_sh.at[pl.ds(row0, CH)], ssem[0]).wait()
    pltpu.make_async_copy(
        zsrc.at[pl.ds(0, ztail)],
        acc_sh.at[pl.ds(row0 + rows_per_tile - ztail, ztail)],
        ssem[1]).wait()
    plsc.subcore_barrier()

    NBUF = 4
    bufs = [rowb.at[b] for b in range(NBUF)]
    ngrp = nch // NBUF

    for b in range(NBUF):
      pltpu.async_copy(table_hbm.at[srcb.at[b]], bufs[b], gsem[b])

    def body(k, carry):
      c = NBUF * k
      for b in range(NBUF):
        pltpu.make_async_copy(
            table_hbm.at[srcb.at[c + b]], bufs[b], gsem[b]).wait()
        pltpu.async_copy(bufs[b], acc_sh.at[dstb.at[c + b]], ssem[b],
                         add=True)
        pltpu.make_async_copy(bufs[b], acc_sh.at[dstb.at[c + b]],
                              ssem[b]).wait()
        pltpu.async_copy(table_hbm.at[srcb.at[c + NBUF + b]], bufs[b],
                         gsem[b])
      return carry

    lax.fori_loop(0, ngrp - 1, body, 0)
    cl = nch - NBUF
    for b in range(NBUF):
      pltpu.make_async_copy(
          table_hbm.at[srcb.at[cl + b]], bufs[b], gsem[b]).wait()
      pltpu.async_copy(bufs[b], acc_sh.at[dstb.at[cl + b]], ssem[b],
                       add=True)
    for b in range(NBUF):
      pltpu.make_async_copy(bufs[b], acc_sh.at[dstb.at[cl + b]],
                            ssem[b]).wait()
    plsc.subcore_barrier()

    pltpu.sync_copy(acc_sh.at[pl.ds(row0, rows_per_tile)],
                    out_hbm.at[cid, pl.ds(row0, rows_per_tile)])

  return scat_kernel(table, src2d, dst2d)


def _tc1(x, w1, disb, n, blk=1000):
  d = w1.shape[1]

  def body(x_ref, w_ref, disb_ref, g1_ref, h1s_ref):
    g = jnp.dot(x_ref[...], w_ref[...], preferred_element_type=jnp.float32)
    g1_ref[...] = g
    h1s_ref[...] = g * disb_ref[...]

  return pl.pallas_call(
      body,
      grid=(n // blk,),
      in_specs=[
          pl.BlockSpec((blk, x.shape[1]), lambda i: (i, 0)),
          pl.BlockSpec(w1.shape, lambda i: (0, 0)),
          pl.BlockSpec((blk, d), lambda i: (i, 0)),
      ],
      out_specs=[
          pl.BlockSpec((blk, d), lambda i: (i, 0)),
          pl.BlockSpec((blk, d), lambda i: (i, 0)),
      ],
      out_shape=[
          jax.ShapeDtypeStruct((n, d), jnp.float32),
          jax.ShapeDtypeStruct((n, d), jnp.float32),
      ],
  )(x, w1, disb)


def _tc2(s1a, s1b, g1, disb, b1, w2, n, blk=1000):
  d = w2.shape[1]

  def body(sa_ref, sb_ref, g1_ref, disb_ref, b1_ref, w2_ref, g2_ref, h2s_ref):
    dis = disb_ref[...]
    h = jnp.maximum(
        dis * (sa_ref[...] + sb_ref[...]) + dis * dis * g1_ref[...]
        + b1_ref[...], 0.0)
    g2 = jnp.dot(h, w2_ref[...], preferred_element_type=jnp.float32)
    g2_ref[...] = g2
    h2s_ref[...] = g2 * dis

  return pl.pallas_call(
      body,
      grid=(n // blk,),
      in_specs=[
          pl.BlockSpec((blk, d), lambda i: (i, 0)),
          pl.BlockSpec((blk, d), lambda i: (i, 0)),
          pl.BlockSpec((blk, d), lambda i: (i, 0)),
          pl.BlockSpec((blk, d), lambda i: (i, 0)),
          pl.BlockSpec((1, d), lambda i: (0, 0)),
          pl.BlockSpec(w2.shape, lambda i: (0, 0)),
      ],
      out_specs=[
          pl.BlockSpec((blk, d), lambda i: (i, 0)),
          pl.BlockSpec((blk, d), lambda i: (i, 0)),
      ],
      out_shape=[
          jax.ShapeDtypeStruct((n, d), jnp.float32),
          jax.ShapeDtypeStruct((n, d), jnp.float32),
      ],
  )(s1a, s1b, g1, disb, b1, w2)


def _tc3(s2a, s2b, g2, disb, b2, x, wp1a, wp1b, bp1, wp2, bp2, n, blk=1000):
  d = g2.shape[1]
  din = x.shape[1]
  pout = wp2.shape[1]

  def body(sa_ref, sb_ref, g2_ref, disb_ref, b2_ref, x_ref,
           wa_ref, wb_ref, bp1_ref, wp2_ref, bp2_ref, y_ref):
    dis = disb_ref[...]
    gnn = (dis * (sa_ref[...] + sb_ref[...]) + dis * dis * g2_ref[...]
           + b2_ref[...])
    p = jnp.maximum(
        jnp.dot(gnn, wa_ref[...], preferred_element_type=jnp.float32)
        + jnp.dot(x_ref[...], wb_ref[...], preferred_element_type=jnp.float32)
        + bp1_ref[...], 0.0)
    y_ref[...] = (jnp.dot(p, wp2_ref[...], preferred_element_type=jnp.float32)
                  + bp2_ref[...])

  return pl.pallas_call(
      body,
      grid=(n // blk,),
      in_specs=[
          pl.BlockSpec((blk, d), lambda i: (i, 0)),
          pl.BlockSpec((blk, d), lambda i: (i, 0)),
          pl.BlockSpec((blk, d), lambda i: (i, 0)),
          pl.BlockSpec((blk, d), lambda i: (i, 0)),
          pl.BlockSpec((1, d), lambda i: (0, 0)),
          pl.BlockSpec((blk, din), lambda i: (i, 0)),
          pl.BlockSpec(wp1a.shape, lambda i: (0, 0)),
          pl.BlockSpec(wp1b.shape, lambda i: (0, 0)),
          pl.BlockSpec((1, wp1a.shape[1]), lambda i: (0, 0)),
          pl.BlockSpec(wp2.shape, lambda i: (0, 0)),
          pl.BlockSpec((1, pout), lambda i: (0, 0)),
      ],
      out_specs=pl.BlockSpec((blk, pout), lambda i: (i, 0)),
      out_shape=jax.ShapeDtypeStruct((n, pout), jnp.float32),
  )(s2a, s2b, g2, disb, b2, x, wp1a, wp1b, bp1, wp2, bp2)


def kernel(x, edge_index, W1, b1, W2, b2, Wp1, bp1, Wp2, bp2):
  n, din = x.shape
  e = edge_index.shape[1]
  d = W1.shape[1]

  src2d = edge_index[0].reshape(e // CH, CH)
  dst2d = edge_index[1].reshape(e // CH, CH)

  deg_parts = _sc_degree(edge_index[1], n)
  dis = lax.rsqrt(1.0 + jnp.sum(deg_parts, axis=0))
  disb = jnp.broadcast_to(dis[:, None], (n, d))

  g1, h1s = _tc1(x, W1, disb, n)
  s1 = _sc_scatter_rows(h1s, src2d, dst2d, n)
  g2, h2s = _tc2(s1[0], s1[1], g1, disb, b1.reshape(1, d), W2, n)
  s2 = _sc_scatter_rows(h2s, src2d, dst2d, n)
  y = _tc3(s2[0], s2[1], g2, disb, b2.reshape(1, d), x,
           Wp1[:d], Wp1[d:], bp1.reshape(1, -1), Wp2, bp2.reshape(1, -1), n)
  return y


# speedup vs baseline: 1.0088x; 1.0055x over previous
import functools

import jax
import jax.numpy as jnp
from jax import lax
from jax.experimental import pallas as pl
from jax.experimental.pallas import tpu as pltpu
from jax.experimental.pallas import tpu_sc as plsc

NC = 2
NS = 16
NW = NC * NS
LANES = 16
CH = 50


def _sc_degree(dst, n):
  e = dst.shape[0]
  epw = e // NW
  mesh = plsc.VectorSubcoreMesh(core_axis_name="c", subcore_axis_name="s")

  @functools.partial(
      pl.kernel,
      mesh=mesh,
      out_type=jax.ShapeDtypeStruct((NW, n), jnp.float32),
      compiler_params=pltpu.CompilerParams(needs_layout_passes=False, use_tc_tiling_on_sc=False),
      scratch_types=[
          pltpu.VMEM((n,), jnp.float32),
          pltpu.VMEM((epw,), jnp.int32),
      ],
  )
  def deg_kernel(dst_hbm, out_hbm, degbuf, dstbuf):
    cid = lax.axis_index("c")
    sid = lax.axis_index("s")
    wid = sid * NC + cid

    zero = jnp.zeros((LANES,), jnp.float32)

    def zbody(i, carry):
      degbuf[pl.ds(i * LANES, LANES)] = zero
      return carry

    lax.fori_loop(0, n // LANES, zbody, 0)

    pltpu.sync_copy(dst_hbm.at[pl.ds(wid * epw, epw)], dstbuf)

    ones = jnp.ones((LANES,), jnp.float32)

    def body(j, carry):
      idx = dstbuf[pl.ds(j * LANES, LANES)]
      plsc.addupdate_scatter(degbuf, [idx], ones)
      return carry

    lax.fori_loop(0, epw // LANES, body, 0)
    pltpu.sync_copy(degbuf, out_hbm.at[wid])

  return deg_kernel(dst)


def _sc_scatter_rows(table, src2d, dst2d, n):
  d = table.shape[1]
  nch_total = src2d.shape[0]
  nch = nch_total // NW
  rows_per_tile = n // NS
  mesh = plsc.VectorSubcoreMesh(core_axis_name="c", subcore_axis_name="s")

  @functools.partial(
      pl.kernel,
      mesh=mesh,
      out_type=jax.ShapeDtypeStruct((NC, n, d), jnp.float32),
      compiler_params=pltpu.CompilerParams(needs_layout_passes=False, use_tc_tiling_on_sc=False),
      scratch_types=[
          pltpu.VMEM_SHARED((n, d), jnp.float32),
          pltpu.VMEM((nch, CH), jnp.int32),
          pltpu.VMEM((nch, CH), jnp.int32),
          pltpu.VMEM((4, CH, d), jnp.float32),
          [pltpu.SemaphoreType.DMA] * 4,
          [pltpu.SemaphoreType.DMA] * 4,
      ],
  )
  def scat_kernel(table_hbm, src_hbm, dst_hbm, out_hbm,
                  acc_sh, srcb, dstb, rowb, gsem, ssem):
    cid = lax.axis_index("c")
    sid = lax.axis_index("s")
    wid = sid * NC + cid

    zero = jnp.zeros((LANES,), jnp.float32)
    zsrc = rowb.at[0]

    def zbody(r, carry):
      for c in range(d // LANES):
        zsrc[r, pl.ds(c * LANES, LANES)] = zero
      return carry

    lax.fori_loop(0, CH, zbody, 0)

    row0 = sid * rows_per_tile
    ztail = rows_per_tile % CH
    for k in range(rows_per_tile // CH):
      pltpu.async_copy(zsrc, acc_sh.at[pl.ds(row0 + k * CH, CH)], ssem[0])
    pltpu.async_copy(zsrc.at[pl.ds(0, ztail)],
                     acc_sh.at[pl.ds(row0 + rows_per_tile - ztail, ztail)],
                     ssem[1])

    pltpu.sync_copy(src_hbm.at[pl.ds(wid * nch, nch)], srcb)
    pltpu.sync_copy(dst_hbm.at[pl.ds(wid * nch, nch)], dstb)

    for k in range(rows_per_tile // CH):
      pltpu.make_async_copy(zsrc, acc_sh.at[pl.ds(row0, CH)], ssem[0]).wait()
    pltpu.make_async_copy(
        zsrc.at[pl.ds(0, ztail)],
        acc_sh.at[pl.ds(row0 + rows_per_tile - ztail, ztail)],
        ssem[1]).wait()
    plsc.subcore_barrier()

    NBUF = 4
    bufs = [rowb.at[b] for b in range(NBUF)]
    ngrp = nch // NBUF

    for b in range(NBUF):
      pltpu.async_copy(table_hbm.at[srcb.at[b]], bufs[b], gsem[b])

    def body(k, carry):
      c = NBUF * k
      for b in range(NBUF):
        pltpu.make_async_copy(
            table_hbm.at[srcb.at[c + b]], bufs[b], gsem[b]).wait()
        pltpu.async_copy(bufs[b], acc_sh.at[dstb.at[c + b]], ssem[b],
                         add=True)
        pltpu.make_async_copy(bufs[b], acc_sh.at[dstb.at[c + b]],
                              ssem[b]).wait()
        pltpu.async_copy(table_hbm.at[srcb.at[c + NBUF + b]], bufs[b],
                         gsem[b])
      return carry

    lax.fori_loop(0, ngrp - 1, body, 0)
    cl = nch - NBUF
    for b in range(NBUF):
      pltpu.make_async_copy(
          table_hbm.at[srcb.at[cl + b]], bufs[b], gsem[b]).wait()
      pltpu.async_copy(bufs[b], acc_sh.at[dstb.at[cl + b]], ssem[b],
                       add=True)
    for b in range(NBUF):
      pltpu.make_async_copy(bufs[b], acc_sh.at[dstb.at[cl + b]],
                            ssem[b]).wait()
    plsc.subcore_barrier()

    pltpu.sync_copy(acc_sh.at[pl.ds(row0, rows_per_tile)],
                    out_hbm.at[cid, pl.ds(row0, rows_per_tile)])

  return scat_kernel(table, src2d, dst2d)


def _tc1(x, w1, disc, n, blk=1000):
  d = w1.shape[1]

  def body(x_ref, w_ref, disc_ref, h1s_ref):
    g = jnp.dot(x_ref[...], w_ref[...], preferred_element_type=jnp.float32)
    h1s_ref[...] = g * disc_ref[...]

  return pl.pallas_call(
      body,
      grid=(n // blk,),
      in_specs=[
          pl.BlockSpec((blk, x.shape[1]), lambda i: (i, 0)),
          pl.BlockSpec(w1.shape, lambda i: (0, 0)),
          pl.BlockSpec((blk, 1), lambda i: (i, 0)),
      ],
      out_specs=pl.BlockSpec((blk, d), lambda i: (i, 0)),
      out_shape=jax.ShapeDtypeStruct((n, d), jnp.float32),
  )(x, w1, disc)


def _tc2(s1a, s1b, h1s, disc, b1, w2, n, blk=1000):
  d = w2.shape[1]

  def body(sa_ref, sb_ref, h1s_ref, disc_ref, b1_ref, w2_ref, h2s_ref):
    dis = disc_ref[...]
    h = jnp.maximum(
        dis * (sa_ref[...] + sb_ref[...] + h1s_ref[...]) + b1_ref[...], 0.0)
    g2 = jnp.dot(h, w2_ref[...], preferred_element_type=jnp.float32)
    h2s_ref[...] = g2 * dis

  return pl.pallas_call(
      body,
      grid=(n // blk,),
      in_specs=[
          pl.BlockSpec((blk, d), lambda i: (i, 0)),
          pl.BlockSpec((blk, d), lambda i: (i, 0)),
          pl.BlockSpec((blk, d), lambda i: (i, 0)),
          pl.BlockSpec((blk, 1), lambda i: (i, 0)),
          pl.BlockSpec((1, d), lambda i: (0, 0)),
          pl.BlockSpec(w2.shape, lambda i: (0, 0)),
      ],
      out_specs=pl.BlockSpec((blk, d), lambda i: (i, 0)),
      out_shape=jax.ShapeDtypeStruct((n, d), jnp.float32),
  )(s1a, s1b, h1s, disc, b1, w2)


def _tc3(s2a, s2b, h2s, disc, b2, x, wp1a, wp1b, bp1, wp2, bp2, n, blk=1000):
  d = h2s.shape[1]
  din = x.shape[1]
  pout = wp2.shape[1]

  def body(sa_ref, sb_ref, h2s_ref, disc_ref, b2_ref, x_ref,
           wa_ref, wb_ref, bp1_ref, wp2_ref, bp2_ref, y_ref):
    dis = disc_ref[...]
    gnn = (dis * (sa_ref[...] + sb_ref[...] + h2s_ref[...])
           + b2_ref[...])
    p = jnp.maximum(
        jnp.dot(gnn, wa_ref[...], preferred_element_type=jnp.float32)
        + jnp.dot(x_ref[...], wb_ref[...], preferred_element_type=jnp.float32)
        + bp1_ref[...], 0.0)
    y_ref[...] = (jnp.dot(p, wp2_ref[...], preferred_element_type=jnp.float32)
                  + bp2_ref[...])

  return pl.pallas_call(
      body,
      grid=(n // blk,),
      in_specs=[
          pl.BlockSpec((blk, d), lambda i: (i, 0)),
          pl.BlockSpec((blk, d), lambda i: (i, 0)),
          pl.BlockSpec((blk, d), lambda i: (i, 0)),
          pl.BlockSpec((blk, 1), lambda i: (i, 0)),
          pl.BlockSpec((1, d), lambda i: (0, 0)),
          pl.BlockSpec((blk, din), lambda i: (i, 0)),
          pl.BlockSpec(wp1a.shape, lambda i: (0, 0)),
          pl.BlockSpec(wp1b.shape, lambda i: (0, 0)),
          pl.BlockSpec((1, wp1a.shape[1]), lambda i: (0, 0)),
          pl.BlockSpec(wp2.shape, lambda i: (0, 0)),
          pl.BlockSpec((1, pout), lambda i: (0, 0)),
      ],
      out_specs=pl.BlockSpec((blk, pout), lambda i: (i, 0)),
      out_shape=jax.ShapeDtypeStruct((n, pout), jnp.float32),
  )(s2a, s2b, h2s, disc, b2, x, wp1a, wp1b, bp1, wp2, bp2)


def kernel(x, edge_index, W1, b1, W2, b2, Wp1, bp1, Wp2, bp2):
  n, din = x.shape
  e = edge_index.shape[1]
  d = W1.shape[1]

  src2d = edge_index[0].reshape(e // CH, CH)
  dst2d = edge_index[1].reshape(e // CH, CH)

  deg_parts = _sc_degree(edge_index[1], n)
  disc = lax.rsqrt(1.0 + jnp.sum(deg_parts, axis=0))[:, None]

  h1s = _tc1(x, W1, disc, n)
  s1 = _sc_scatter_rows(h1s, src2d, dst2d, n)
  h2s = _tc2(s1[0], s1[1], h1s, disc, b1.reshape(1, d), W2, n)
  s2 = _sc_scatter_rows(h2s, src2d, dst2d, n)
  y = _tc3(s2[0], s2[1], h2s, disc, b2.reshape(1, d), x,
           Wp1[:d], Wp1[d:], bp1.reshape(1, -1), Wp2, bp2.reshape(1, -1), n)
  return y


# speedup vs baseline: 1.0407x; 1.0316x over previous
import functools

import jax
import jax.numpy as jnp
from jax import lax
from jax.experimental import pallas as pl
from jax.experimental.pallas import tpu as pltpu
from jax.experimental.pallas import tpu_sc as plsc

NC = 2
NS = 16
NW = NC * NS
LANES = 16
CH = 50


def _sc_degree(dst, n):
  e = dst.shape[0]
  epw = e // NW
  mesh = plsc.VectorSubcoreMesh(core_axis_name="c", subcore_axis_name="s")

  @functools.partial(
      pl.kernel,
      mesh=mesh,
      out_type=jax.ShapeDtypeStruct((NW, n), jnp.float32),
      compiler_params=pltpu.CompilerParams(needs_layout_passes=False, use_tc_tiling_on_sc=False),
      scratch_types=[
          pltpu.VMEM((n,), jnp.float32),
          pltpu.VMEM((epw,), jnp.int32),
      ],
  )
  def deg_kernel(dst_hbm, out_hbm, degbuf, dstbuf):
    cid = lax.axis_index("c")
    sid = lax.axis_index("s")
    wid = sid * NC + cid

    zero = jnp.zeros((LANES,), jnp.float32)

    def zbody(i, carry):
      degbuf[pl.ds(i * LANES, LANES)] = zero
      return carry

    lax.fori_loop(0, n // LANES, zbody, 0)

    pltpu.sync_copy(dst_hbm.at[pl.ds(wid * epw, epw)], dstbuf)

    ones = jnp.ones((LANES,), jnp.float32)

    def body(j, carry):
      idx = dstbuf[pl.ds(j * LANES, LANES)]
      plsc.addupdate_scatter(degbuf, [idx], ones)
      return carry

    lax.fori_loop(0, epw // LANES, body, 0)
    pltpu.sync_copy(degbuf, out_hbm.at[wid])

  return deg_kernel(dst)


def _sc_scatter_rows(table, src2d, dst2d, n):
  d = table.shape[1]
  dt = table.dtype
  vl = LANES * 4 // dt.itemsize
  nch_total = src2d.shape[0]
  nch = nch_total // NW
  rows_per_tile = n // NS
  mesh = plsc.VectorSubcoreMesh(core_axis_name="c", subcore_axis_name="s")

  @functools.partial(
      pl.kernel,
      mesh=mesh,
      out_type=jax.ShapeDtypeStruct((NC, n, d), dt),
      compiler_params=pltpu.CompilerParams(needs_layout_passes=False, use_tc_tiling_on_sc=False),
      scratch_types=[
          pltpu.VMEM_SHARED((n, d), dt),
          pltpu.VMEM((nch, CH), jnp.int32),
          pltpu.VMEM((nch, CH), jnp.int32),
          pltpu.VMEM((4, CH, d), dt),
          [pltpu.SemaphoreType.DMA] * 4,
          [pltpu.SemaphoreType.DMA] * 4,
      ],
  )
  def scat_kernel(table_hbm, src_hbm, dst_hbm, out_hbm,
                  acc_sh, srcb, dstb, rowb, gsem, ssem):
    cid = lax.axis_index("c")
    sid = lax.axis_index("s")
    wid = sid * NC + cid

    zero = jnp.zeros((vl,), dt)
    zsrc = rowb.at[0]

    def zbody(r, carry):
      for c in range(d // vl):
        zsrc[r, pl.ds(c * vl, vl)] = zero
      return carry

    lax.fori_loop(0, CH, zbody, 0)

    row0 = sid * rows_per_tile
    ztail = rows_per_tile % CH
    for k in range(rows_per_tile // CH):
      pltpu.async_copy(zsrc, acc_sh.at[pl.ds(row0 + k * CH, CH)], ssem[0])
    pltpu.async_copy(zsrc.at[pl.ds(0, ztail)],
                     acc_sh.at[pl.ds(row0 + rows_per_tile - ztail, ztail)],
                     ssem[1])

    pltpu.sync_copy(src_hbm.at[pl.ds(wid * nch, nch)], srcb)
    pltpu.sync_copy(dst_hbm.at[pl.ds(wid * nch, nch)], dstb)

    for k in range(rows_per_tile // CH):
      pltpu.make_async_copy(zsrc, acc_sh.at[pl.ds(row0, CH)], ssem[0]).wait()
    pltpu.make_async_copy(
        zsrc.at[pl.ds(0, ztail)],
        acc_sh.at[pl.ds(row0 + rows_per_tile - ztail, ztail)],
        ssem[1]).wait()
    plsc.subcore_barrier()

    NBUF = 4
    bufs = [rowb.at[b] for b in range(NBUF)]
    ngrp = nch // NBUF

    for b in range(NBUF):
      pltpu.async_copy(table_hbm.at[srcb.at[b]], bufs[b], gsem[b])

    def body(k, carry):
      c = NBUF * k
      for b in range(NBUF):
        pltpu.make_async_copy(
            table_hbm.at[srcb.at[c + b]], bufs[b], gsem[b]).wait()
        pltpu.async_copy(bufs[b], acc_sh.at[dstb.at[c + b]], ssem[b],
                         add=True)
        pltpu.make_async_copy(bufs[b], acc_sh.at[dstb.at[c + b]],
                              ssem[b]).wait()
        pltpu.async_copy(table_hbm.at[srcb.at[c + NBUF + b]], bufs[b],
                         gsem[b])
      return carry

    lax.fori_loop(0, ngrp - 1, body, 0)
    cl = nch - NBUF
    for b in range(NBUF):
      pltpu.make_async_copy(
          table_hbm.at[srcb.at[cl + b]], bufs[b], gsem[b]).wait()
      pltpu.async_copy(bufs[b], acc_sh.at[dstb.at[cl + b]], ssem[b],
                       add=True)
    for b in range(NBUF):
      pltpu.make_async_copy(bufs[b], acc_sh.at[dstb.at[cl + b]],
                            ssem[b]).wait()
    plsc.subcore_barrier()

    pltpu.sync_copy(acc_sh.at[pl.ds(row0, rows_per_tile)],
                    out_hbm.at[cid, pl.ds(row0, rows_per_tile)])

  return scat_kernel(table, src2d, dst2d)


def _tc1(x, w1, disc, n, blk=1000):
  d = w1.shape[1]

  def body(x_ref, w_ref, disc_ref, h1s_ref):
    g = jnp.dot(x_ref[...], w_ref[...], preferred_element_type=jnp.float32)
    h1s_ref[...] = (g * disc_ref[...]).astype(jnp.bfloat16)

  return pl.pallas_call(
      body,
      grid=(n // blk,),
      in_specs=[
          pl.BlockSpec((blk, x.shape[1]), lambda i: (i, 0)),
          pl.BlockSpec(w1.shape, lambda i: (0, 0)),
          pl.BlockSpec((blk, 1), lambda i: (i, 0)),
      ],
      out_specs=pl.BlockSpec((blk, d), lambda i: (i, 0)),
      out_shape=jax.ShapeDtypeStruct((n, d), jnp.bfloat16),
  )(x, w1, disc)


def _tc2(s1a, s1b, h1s, disc, b1, w2, n, blk=1000):
  d = w2.shape[1]

  def body(sa_ref, sb_ref, h1s_ref, disc_ref, b1_ref, w2_ref, h2s_ref):
    dis = disc_ref[...]
    s = (sa_ref[...].astype(jnp.float32) + sb_ref[...].astype(jnp.float32)
         + h1s_ref[...].astype(jnp.float32))
    h = jnp.maximum(dis * s + b1_ref[...], 0.0)
    g2 = jnp.dot(h, w2_ref[...], preferred_element_type=jnp.float32)
    h2s_ref[...] = (g2 * dis).astype(jnp.bfloat16)

  return pl.pallas_call(
      body,
      grid=(n // blk,),
      in_specs=[
          pl.BlockSpec((blk, d), lambda i: (i, 0)),
          pl.BlockSpec((blk, d), lambda i: (i, 0)),
          pl.BlockSpec((blk, d), lambda i: (i, 0)),
          pl.BlockSpec((blk, 1), lambda i: (i, 0)),
          pl.BlockSpec((1, d), lambda i: (0, 0)),
          pl.BlockSpec(w2.shape, lambda i: (0, 0)),
      ],
      out_specs=pl.BlockSpec((blk, d), lambda i: (i, 0)),
      out_shape=jax.ShapeDtypeStruct((n, d), jnp.bfloat16),
  )(s1a, s1b, h1s, disc, b1, w2)


def _tc3(s2a, s2b, h2s, disc, b2, x, wp1a, wp1b, bp1, wp2, bp2, n, blk=1000):
  d = h2s.shape[1]
  din = x.shape[1]
  pout = wp2.shape[1]

  def body(sa_ref, sb_ref, h2s_ref, disc_ref, b2_ref, x_ref,
           wa_ref, wb_ref, bp1_ref, wp2_ref, bp2_ref, y_ref):
    dis = disc_ref[...]
    s = (sa_ref[...].astype(jnp.float32) + sb_ref[...].astype(jnp.float32)
         + h2s_ref[...].astype(jnp.float32))
    gnn = dis * s + b2_ref[...]
    p = jnp.maximum(
        jnp.dot(gnn, wa_ref[...], preferred_element_type=jnp.float32)
        + jnp.dot(x_ref[...], wb_ref[...], preferred_element_type=jnp.float32)
        + bp1_ref[...], 0.0)
    y_ref[...] = (jnp.dot(p, wp2_ref[...], preferred_element_type=jnp.float32)
                  + bp2_ref[...])

  return pl.pallas_call(
      body,
      grid=(n // blk,),
      in_specs=[
          pl.BlockSpec((blk, d), lambda i: (i, 0)),
          pl.BlockSpec((blk, d), lambda i: (i, 0)),
          pl.BlockSpec((blk, d), lambda i: (i, 0)),
          pl.BlockSpec((blk, 1), lambda i: (i, 0)),
          pl.BlockSpec((1, d), lambda i: (0, 0)),
          pl.BlockSpec((blk, din), lambda i: (i, 0)),
          pl.BlockSpec(wp1a.shape, lambda i: (0, 0)),
          pl.BlockSpec(wp1b.shape, lambda i: (0, 0)),
          pl.BlockSpec((1, wp1a.shape[1]), lambda i: (0, 0)),
          pl.BlockSpec(wp2.shape, lambda i: (0, 0)),
          pl.BlockSpec((1, pout), lambda i: (0, 0)),
      ],
      out_specs=pl.BlockSpec((blk, pout), lambda i: (i, 0)),
      out_shape=jax.ShapeDtypeStruct((n, pout), jnp.float32),
  )(s2a, s2b, h2s, disc, b2, x, wp1a, wp1b, bp1, wp2, bp2)


def kernel(x, edge_index, W1, b1, W2, b2, Wp1, bp1, Wp2, bp2):
  n, din = x.shape
  e = edge_index.shape[1]
  d = W1.shape[1]

  src2d = edge_index[0].reshape(e // CH, CH)
  dst2d = edge_index[1].reshape(e // CH, CH)

  deg_parts = _sc_degree(edge_index[1], n)
  disc = lax.rsqrt(1.0 + jnp.sum(deg_parts, axis=0))[:, None]

  h1s = _tc1(x, W1, disc, n)
  s1 = _sc_scatter_rows(h1s, src2d, dst2d, n)
  h2s = _tc2(s1[0], s1[1], h1s, disc, b1.reshape(1, d), W2, n)
  s2 = _sc_scatter_rows(h2s, src2d, dst2d, n)
  y = _tc3(s2[0], s2[1], h2s, disc, b2.reshape(1, d), x,
           Wp1[:d], Wp1[d:], bp1.reshape(1, -1), Wp2, bp2.reshape(1, -1), n)
  return y


# speedup vs baseline: 1.0706x; 1.0287x over previous
import functools

import jax
import jax.numpy as jnp
from jax import lax
from jax.experimental import pallas as pl
from jax.experimental.pallas import tpu as pltpu
from jax.experimental.pallas import tpu_sc as plsc

NC = 2
NS = 16
NW = NC * NS
LANES = 16
CH = 50


def _sc_degree(dst, n):
  e = dst.shape[0]
  epw = e // NW
  mesh = plsc.VectorSubcoreMesh(core_axis_name="c", subcore_axis_name="s")

  @functools.partial(
      pl.kernel,
      mesh=mesh,
      out_type=jax.ShapeDtypeStruct((NW, n), jnp.float32),
      compiler_params=pltpu.CompilerParams(needs_layout_passes=False, use_tc_tiling_on_sc=False),
      scratch_types=[
          pltpu.VMEM((n,), jnp.float32),
          pltpu.VMEM((epw,), jnp.int32),
      ],
  )
  def deg_kernel(dst_hbm, out_hbm, degbuf, dstbuf):
    cid = lax.axis_index("c")
    sid = lax.axis_index("s")
    wid = sid * NC + cid

    zero = jnp.zeros((LANES,), jnp.float32)

    def zbody(i, carry):
      degbuf[pl.ds(i * LANES, LANES)] = zero
      return carry

    lax.fori_loop(0, n // LANES, zbody, 0)

    pltpu.sync_copy(dst_hbm.at[pl.ds(wid * epw, epw)], dstbuf)

    ones = jnp.ones((LANES,), jnp.float32)

    def body(j, carry):
      idx = dstbuf[pl.ds(j * LANES, LANES)]
      plsc.addupdate_scatter(degbuf, [idx], ones)
      return carry

    lax.fori_loop(0, epw // LANES, body, 0)
    pltpu.sync_copy(degbuf, out_hbm.at[wid])

  return deg_kernel(dst)


def _sc_scatter_rows(table, src2d, dst2d, n):
  d = table.shape[1]
  dt = table.dtype
  vl = LANES * 4 // dt.itemsize
  nch_total = src2d.shape[0]
  nch = nch_total // NW
  rows_per_tile = n // NS
  mesh = plsc.VectorSubcoreMesh(core_axis_name="c", subcore_axis_name="s")

  @functools.partial(
      pl.kernel,
      mesh=mesh,
      out_type=jax.ShapeDtypeStruct((NC, n, d), dt),
      compiler_params=pltpu.CompilerParams(needs_layout_passes=False, use_tc_tiling_on_sc=False),
      scratch_types=[
          pltpu.VMEM_SHARED((n, d), dt),
          pltpu.VMEM((nch, CH), jnp.int32),
          pltpu.VMEM((nch, CH), jnp.int32),
          pltpu.VMEM((4, CH, d), dt),
          [pltpu.SemaphoreType.DMA] * 4,
          [pltpu.SemaphoreType.DMA] * 4,
      ],
  )
  def scat_kernel(table_hbm, src_hbm, dst_hbm, out_hbm,
                  acc_sh, srcb, dstb, rowb, gsem, ssem):
    cid = lax.axis_index("c")
    sid = lax.axis_index("s")
    wid = sid * NC + cid

    zero = jnp.zeros((vl,), dt)
    zsrc = rowb.at[0]

    def zbody(r, carry):
      for c in range(d // vl):
        zsrc[r, pl.ds(c * vl, vl)] = zero
      return carry

    lax.fori_loop(0, CH, zbody, 0)

    row0 = sid * rows_per_tile
    ztail = rows_per_tile % CH
    for k in range(rows_per_tile // CH):
      pltpu.async_copy(zsrc, acc_sh.at[pl.ds(row0 + k * CH, CH)], ssem[0])
    pltpu.async_copy(zsrc.at[pl.ds(0, ztail)],
                     acc_sh.at[pl.ds(row0 + rows_per_tile - ztail, ztail)],
                     ssem[1])

    pltpu.sync_copy(src_hbm.at[pl.ds(wid * nch, nch)], srcb)
    pltpu.sync_copy(dst_hbm.at[pl.ds(wid * nch, nch)], dstb)

    for k in range(rows_per_tile // CH):
      pltpu.make_async_copy(zsrc, acc_sh.at[pl.ds(row0, CH)], ssem[0]).wait()
    pltpu.make_async_copy(
        zsrc.at[pl.ds(0, ztail)],
        acc_sh.at[pl.ds(row0 + rows_per_tile - ztail, ztail)],
        ssem[1]).wait()
    plsc.subcore_barrier()

    NBUF = 4
    bufs = [rowb.at[b] for b in range(NBUF)]
    ngrp = nch // NBUF

    for b in range(NBUF):
      pltpu.async_copy(table_hbm.at[srcb.at[b]], bufs[b], gsem[b])

    def body(k, carry):
      c = NBUF * k
      for b in range(NBUF):
        pltpu.make_async_copy(
            table_hbm.at[srcb.at[c + b]], bufs[b], gsem[b]).wait()
        pltpu.async_copy(bufs[b], acc_sh.at[dstb.at[c + b]], ssem[b],
                         add=True)
        pltpu.make_async_copy(bufs[b], acc_sh.at[dstb.at[c + b]],
                              ssem[b]).wait()
        pltpu.async_copy(table_hbm.at[srcb.at[c + NBUF + b]], bufs[b],
                         gsem[b])
      return carry

    lax.fori_loop(0, ngrp - 1, body, 0)
    cl = nch - NBUF
    for b in range(NBUF):
      pltpu.make_async_copy(
          table_hbm.at[srcb.at[cl + b]], bufs[b], gsem[b]).wait()
      pltpu.async_copy(bufs[b], acc_sh.at[dstb.at[cl + b]], ssem[b],
                       add=True)
    for b in range(NBUF):
      pltpu.make_async_copy(bufs[b], acc_sh.at[dstb.at[cl + b]],
                            ssem[b]).wait()
    plsc.subcore_barrier()

    pltpu.sync_copy(acc_sh.at[pl.ds(row0, rows_per_tile)],
                    out_hbm.at[cid, pl.ds(row0, rows_per_tile)])

  return scat_kernel(table, src2d, dst2d)


def _tc1(x, w1, disc, n, blk=2000):
  d = w1.shape[1]

  def body(x_ref, w_ref, disc_ref, h1s_ref):
    g = jnp.dot(x_ref[...], w_ref[...], preferred_element_type=jnp.float32)
    h1s_ref[...] = (g * disc_ref[...]).astype(jnp.bfloat16)

  return pl.pallas_call(
      body,
      grid=(n // blk,),
      in_specs=[
          pl.BlockSpec((blk, x.shape[1]), lambda i: (i, 0)),
          pl.BlockSpec(w1.shape, lambda i: (0, 0)),
          pl.BlockSpec((blk, 1), lambda i: (i, 0)),
      ],
      out_specs=pl.BlockSpec((blk, d), lambda i: (i, 0)),
      out_shape=jax.ShapeDtypeStruct((n, d), jnp.bfloat16),
  )(x, w1, disc)


def _tc2(s1a, s1b, h1s, disc, b1, w2, n, blk=2000):
  d = w2.shape[1]

  def body(sa_ref, sb_ref, h1s_ref, disc_ref, b1_ref, w2_ref, h2s_ref):
    dis = disc_ref[...]
    s = (sa_ref[...].astype(jnp.float32) + sb_ref[...].astype(jnp.float32)
         + h1s_ref[...].astype(jnp.float32))
    h = jnp.maximum(dis * s + b1_ref[...], 0.0)
    g2 = jnp.dot(h, w2_ref[...], preferred_element_type=jnp.float32)
    h2s_ref[...] = (g2 * dis).astype(jnp.bfloat16)

  return pl.pallas_call(
      body,
      grid=(n // blk,),
      in_specs=[
          pl.BlockSpec((blk, d), lambda i: (i, 0)),
          pl.BlockSpec((blk, d), lambda i: (i, 0)),
          pl.BlockSpec((blk, d), lambda i: (i, 0)),
          pl.BlockSpec((blk, 1), lambda i: (i, 0)),
          pl.BlockSpec((1, d), lambda i: (0, 0)),
          pl.BlockSpec(w2.shape, lambda i: (0, 0)),
      ],
      out_specs=pl.BlockSpec((blk, d), lambda i: (i, 0)),
      out_shape=jax.ShapeDtypeStruct((n, d), jnp.bfloat16),
  )(s1a, s1b, h1s, disc, b1, w2)


def _tc3(s2a, s2b, h2s, disc, b2, x, wp1a, wp1b, bp1, wp2, bp2, n, blk=2000):
  d = h2s.shape[1]
  din = x.shape[1]
  pout = wp2.shape[1]

  def body(sa_ref, sb_ref, h2s_ref, disc_ref, b2_ref, x_ref,
           wa_ref, wb_ref, bp1_ref, wp2_ref, bp2_ref, y_ref):
    dis = disc_ref[...]
    s = (sa_ref[...].astype(jnp.float32) + sb_ref[...].astype(jnp.float32)
         + h2s_ref[...].astype(jnp.float32))
    gnn = dis * s + b2_ref[...]
    p = jnp.maximum(
        jnp.dot(gnn, wa_ref[...], preferred_element_type=jnp.float32)
        + jnp.dot(x_ref[...], wb_ref[...], preferred_element_type=jnp.float32)
        + bp1_ref[...], 0.0)
    y_ref[...] = (jnp.dot(p, wp2_ref[...], preferred_element_type=jnp.float32)
                  + bp2_ref[...])

  return pl.pallas_call(
      body,
      grid=(n // blk,),
      in_specs=[
          pl.BlockSpec((blk, d), lambda i: (i, 0)),
          pl.BlockSpec((blk, d), lambda i: (i, 0)),
          pl.BlockSpec((blk, d), lambda i: (i, 0)),
          pl.BlockSpec((blk, 1), lambda i: (i, 0)),
          pl.BlockSpec((1, d), lambda i: (0, 0)),
          pl.BlockSpec((blk, din), lambda i: (i, 0)),
          pl.BlockSpec(wp1a.shape, lambda i: (0, 0)),
          pl.BlockSpec(wp1b.shape, lambda i: (0, 0)),
          pl.BlockSpec((1, wp1a.shape[1]), lambda i: (0, 0)),
          pl.BlockSpec(wp2.shape, lambda i: (0, 0)),
          pl.BlockSpec((1, pout), lambda i: (0, 0)),
      ],
      out_specs=pl.BlockSpec((blk, pout), lambda i: (i, 0)),
      out_shape=jax.ShapeDtypeStruct((n, pout), jnp.float32),
  )(s2a, s2b, h2s, disc, b2, x, wp1a, wp1b, bp1, wp2, bp2)


def kernel(x, edge_index, W1, b1, W2, b2, Wp1, bp1, Wp2, bp2):
  n, din = x.shape
  e = edge_index.shape[1]
  d = W1.shape[1]

  src2d = edge_index[0].reshape(e // CH, CH)
  dst2d = edge_index[1].reshape(e // CH, CH)

  deg_parts = _sc_degree(edge_index[1], n)
  disc = lax.rsqrt(1.0 + jnp.sum(deg_parts, axis=0))[:, None]

  h1s = _tc1(x, W1, disc, n)
  s1 = _sc_scatter_rows(h1s, src2d, dst2d, n)
  h2s = _tc2(s1[0], s1[1], h1s, disc, b1.reshape(1, d), W2, n)
  s2 = _sc_scatter_rows(h2s, src2d, dst2d, n)
  y = _tc3(s2[0], s2[1], h2s, disc, b2.reshape(1, d), x,
           Wp1[:d], Wp1[d:], bp1.reshape(1, -1), Wp2, bp2.reshape(1, -1), n)
  return y
